# Initial kernel scaffold; baseline (speedup 1.0000x reference)
import jax
import jax.numpy as jnp
from jax.experimental import pallas as pl


def kernel(x, edge_index, batch_PI, gcn_W0, gcn_b0, mlp0_W1, mlp0_b1, mlp0_W2, mlp0_b2, gcn_W1, gcn_b1, mlp1_W1, mlp1_b1, mlp1_W2, mlp1_b2, conv_W, conv_b, pi_W1, pi_W2, pi_W3, pi_bias, gt_W1, gt_W2, gt_W3, gt_bias, ot_W1, ot_W2, ot_W3, ot_bias, att_W, att_b, out_W, out_b):
    raise NotImplementedError("write your pallas kernel here")


# speedup vs baseline: 11.3045x; 11.3045x over previous
import functools

import jax
import jax.numpy as jnp
from jax import lax
from jax.experimental import pallas as pl
from jax.experimental.pallas import tpu as pltpu
from jax.experimental.pallas import tpu_sc as plsc

_N = 10000
_NPAD = 10240
_E = 160000
_CB = 128
_NCHUNK = _E // _CB
_NW = 32
_CPW = -(-_NCHUNK // _NW)
_RPW = _NPAD // 16
_ZCH = _RPW // _CB
_G = 8
_NPG = 1250
_F32 = jnp.float32


def _sc_mesh():
    return plsc.VectorSubcoreMesh(core_axis_name="c", subcore_axis_name="s")


def _sc_degree(dst):
    @functools.partial(
        pl.kernel,
        out_type=jax.ShapeDtypeStruct((2, _NPAD, 16), _F32),
        mesh=_sc_mesh(),
        compiler_params=pltpu.CompilerParams(use_tc_tiling_on_sc=False),
        scratch_types=[
            pltpu.VMEM((_CB,), jnp.int32),
            pltpu.VMEM((_CB, 16), _F32),
            pltpu.VMEM_SHARED((_NPAD, 16), _F32),
        ],
    )
    def k(dst_hbm, out_hbm, idx_v, val_v, acc_sh):
        c = lax.axis_index("c")
        s = lax.axis_index("s")
        wid = s * 2 + c

        def fill_zero(i, carry):
            val_v[i, :] = jnp.zeros((16,), _F32)
            return carry

        lax.fori_loop(0, _CB, fill_zero, 0)
        for z in range(_ZCH):
            pltpu.sync_copy(val_v, acc_sh.at[pl.ds(s * _RPW + z * _CB, _CB)])

        def fill_one(i, carry):
            val_v[i, :] = jnp.ones((16,), _F32)
            return carry

        lax.fori_loop(0, _CB, fill_one, 0)
        plsc.subcore_barrier()

        def body(kk, carry):
            cid = wid + kk * _NW

            @pl.when(cid < _NCHUNK)
            def _():
                pltpu.sync_copy(dst_hbm.at[pl.ds(cid * _CB, _CB)], idx_v)
                pltpu.sync_copy(val_v, acc_sh.at[idx_v], add=True)

            return carry

        lax.fori_loop(0, _CPW, body, 0)
        plsc.subcore_barrier()
        for z in range(_ZCH):
            pltpu.sync_copy(
                acc_sh.at[pl.ds(s * _RPW + z * _CB, _CB)],
                out_hbm.at[c, pl.ds(s * _RPW + z * _CB, _CB)],
            )

    return k(dst)


def _sc_scatter(p, src, dst):
    @functools.partial(
        pl.kernel,
        out_type=jax.ShapeDtypeStruct((2, _NPAD, 16), _F32),
        mesh=_sc_mesh(),
        compiler_params=pltpu.CompilerParams(use_tc_tiling_on_sc=False),
        scratch_types=[
            pltpu.VMEM((_CB,), jnp.int32),
            pltpu.VMEM((_CB,), jnp.int32),
            pltpu.VMEM((_CB, 16), _F32),
            pltpu.VMEM((_CB, 16), _F32),
            pltpu.VMEM_SHARED((_NPAD, 16), _F32),
            pltpu.SemaphoreType.DMA,
        ],
    )
    def k(p_hbm, src_hbm, dst_hbm, out_hbm, sidx_v, didx_v, rows_v, zero_v,
          acc_sh, sem):
        c = lax.axis_index("c")
        s = lax.axis_index("s")
        wid = s * 2 + c

        def fill_zero(i, carry):
            zero_v[i, :] = jnp.zeros((16,), _F32)
            return carry

        lax.fori_loop(0, _CB, fill_zero, 0)
        for z in range(_ZCH):
            pltpu.sync_copy(zero_v, acc_sh.at[pl.ds(s * _RPW + z * _CB, _CB)])
        plsc.subcore_barrier()

        def body(kk, carry):
            cid = wid + kk * _NW

            @pl.when(cid < _NCHUNK)
            def _():
                pltpu.sync_copy(src_hbm.at[pl.ds(cid * _CB, _CB)], sidx_v)
                pltpu.sync_copy(dst_hbm.at[pl.ds(cid * _CB, _CB)], didx_v)
                pltpu.async_copy(p_hbm.at[sidx_v], rows_v, sem).wait()
                pltpu.sync_copy(rows_v, acc_sh.at[didx_v], add=True)

            return carry

        lax.fori_loop(0, _CPW, body, 0)
        plsc.subcore_barrier()
        for z in range(_ZCH):
            pltpu.sync_copy(
                acc_sh.at[pl.ds(s * _RPW + z * _CB, _CB)],
                out_hbm.at[c, pl.ds(s * _RPW + z * _CB, _CB)],
            )

    return k(p, src, dst)


def _tc_prep0(x3, W0, degp4):
    def body(x_ref, w_ref, degp_ref, p0_ref, dinv_ref):
        deg = degp_ref[0, 0] + degp_ref[1, 0] + 1.0
        dinv = 1.0 / jnp.sqrt(deg[:, 0:1])
        h0 = jnp.dot(x_ref[0], w_ref[...], preferred_element_type=_F32,
                    precision=lax.Precision.HIGHEST)
        p0_ref[0] = h0 * dinv
        dinv_ref[0] = dinv

    return pl.pallas_call(
        body,
        grid=(_G,),
        in_specs=[
            pl.BlockSpec((1, _NPG, 128), lambda g: (g, 0, 0)),
            pl.BlockSpec((128, 16), lambda g: (0, 0)),
            pl.BlockSpec((2, 1, _NPG, 16), lambda g: (0, g, 0, 0)),
        ],
        out_specs=[
            pl.BlockSpec((1, _NPG, 16), lambda g: (g, 0, 0)),
            pl.BlockSpec((1, _NPG, 1), lambda g: (g, 0, 0)),
        ],
        out_shape=[
            jax.ShapeDtypeStruct((_G, _NPG, 16), _F32),
            jax.ShapeDtypeStruct((_G, _NPG, 1), _F32),
        ],
    )(x3, W0, degp4)


def _tc_mid(acc4, p0, dinv, b0, W1, b1, W2, b2, gW1):
    def body(acc_ref, p0_ref, dinv_ref, b0_ref, w1_ref, b1_ref, w2_ref,
             b2_ref, gw1_ref, h1_ref, p1_ref):
        dinv_b = dinv_ref[0]
        g0 = (acc_ref[0, 0] + acc_ref[1, 0] + p0_ref[0]) * dinv_b + b0_ref[...]
        t = jnp.maximum(
            jnp.dot(g0, w1_ref[...], preferred_element_type=_F32,
                    precision=lax.Precision.HIGHEST)
            + b1_ref[...], 0.0)
        h1 = jnp.dot(t, w2_ref[...], preferred_element_type=_F32,
                    precision=lax.Precision.HIGHEST) + b2_ref[...]
        h1_ref[0] = h1
        p1_ref[0] = jnp.dot(h1, gw1_ref[...],
                            preferred_element_type=_F32,
                    precision=lax.Precision.HIGHEST) * dinv_b

    full = lambda shape: pl.BlockSpec(shape, lambda g: tuple(0 for _ in shape))
    return pl.pallas_call(
        body,
        grid=(_G,),
        in_specs=[
            pl.BlockSpec((2, 1, _NPG, 16), lambda g: (0, g, 0, 0)),
            pl.BlockSpec((1, _NPG, 16), lambda g: (g, 0, 0)),
            pl.BlockSpec((1, _NPG, 1), lambda g: (g, 0, 0)),
            full((1, 16)),
            full((16, 16)),
            full((1, 16)),
            full((16, 256)),
            full((1, 256)),
            full((256, 16)),
        ],
        out_specs=[
            pl.BlockSpec((1, _NPG, 256), lambda g: (g, 0, 0)),
            pl.BlockSpec((1, _NPG, 16), lambda g: (g, 0, 0)),
        ],
        out_shape=[
            jax.ShapeDtypeStruct((_G, _NPG, 256), _F32),
            jax.ShapeDtypeStruct((_G, _NPG, 16), _F32),
        ],
    )(acc4, p0, dinv, b0, W1, b1, W2, b2, gW1)


def _tc_final(acc4, p1, dinv, h1, gb1, W1, b1, W2, b2, R, PI3, Wc, cb, up,
              VW, ug, Cc):
    def body(acc_ref, p1_ref, dinv_ref, h1_ref, gb1_ref, w1_ref, b1_ref,
             w2_ref, b2_ref, r_ref, pi_ref, wc_ref, cb_ref, up_ref, vw_ref,
             ug_ref, cc_ref, out_ref):
        dinv_b = dinv_ref[0]
        g1 = (acc_ref[0, 0] + acc_ref[1, 0] + p1_ref[0]) * dinv_b \
            + gb1_ref[...]
        t = jnp.maximum(
            jnp.dot(g1, w1_ref[...], preferred_element_type=_F32,
                    precision=lax.Precision.HIGHEST)
            + b1_ref[...], 0.0)
        h2 = jnp.dot(t, w2_ref[...], preferred_element_type=_F32,
                    precision=lax.Precision.HIGHEST) + b2_ref[...]
        s1 = jnp.sum(h1_ref[0], axis=0, keepdims=True) * (1.0 / _NPG)
        s2 = jnp.sum(h2, axis=0, keepdims=True) * (1.0 / _NPG)
        m1 = jnp.dot(s1, r_ref[...], preferred_element_type=_F32,
                    precision=lax.Precision.HIGHEST)
        m2 = jnp.dot(s2, r_ref[...], preferred_element_type=_F32,
                    precision=lax.Precision.HIGHEST)
        pe = jnp.maximum(
            jnp.dot(pi_ref[0], wc_ref[...], preferred_element_type=_F32,
                    precision=lax.Precision.HIGHEST)
            + cb_ref[...], 0.0)
        peu = jnp.dot(pe, up_ref[...], preferred_element_type=_F32,
                    precision=lax.Precision.HIGHEST)
        sp = jnp.sum(peu * vw_ref[...], axis=0, keepdims=True)
        out_ref[0] = ug_ref[0, 0] * m1 + ug_ref[0, 1] * m2 + sp + cc_ref[...]

    full = lambda shape: pl.BlockSpec(shape, lambda g: tuple(0 for _ in shape))
    return pl.pallas_call(
        body,
        grid=(_G,),
        in_specs=[
            pl.BlockSpec((2, 1, _NPG, 16), lambda g: (0, g, 0, 0)),
            pl.BlockSpec((1, _NPG, 16), lambda g: (g, 0, 0)),
            pl.BlockSpec((1, _NPG, 1), lambda g: (g, 0, 0)),
            pl.BlockSpec((1, _NPG, 256), lambda g: (g, 0, 0)),
            full((1, 16)),
            full((16, 16)),
            full((1, 16)),
            full((16, 256)),
            full((1, 256)),
            full((256, 10)),
            pl.BlockSpec((1, 625, 20), lambda g: (g, 0, 0)),
            full((20, 16)),
            full((1, 16)),
            full((16, 1)),
            full((625, 10)),
            full((1, 2)),
            full((1, 10)),
        ],
        out_specs=pl.BlockSpec((1, 1, 10), lambda g: (g, 0, 0)),
        out_shape=jax.ShapeDtypeStruct((_G, 1, 10), _F32),
    )(acc4, p1, dinv, h1, gb1, W1, b1, W2, b2, R, PI3, Wc, cb, up, VW, ug, Cc)


def kernel(x, edge_index, batch_PI, gcn_W0, gcn_b0, mlp0_W1, mlp0_b1,
           mlp0_W2, mlp0_b2, gcn_W1, gcn_b1, mlp1_W1, mlp1_b1, mlp1_W2,
           mlp1_b2, conv_W, conv_b, pi_W1, pi_W2, pi_W3, pi_bias, gt_W1,
           gt_W2, gt_W3, gt_bias, ot_W1, ot_W2, ot_W3, ot_bias, att_W,
           att_b, out_W, out_b):
    src = edge_index[0]
    dst = edge_index[1]

    u = ot_W1.T @ att_W[:, 0]
    v = ot_W2.T @ out_W
    w = ot_W3.sum(axis=0)
    u1, u2 = u[:16], u[16:]
    ug = gt_W1.T @ u1
    vg = gt_W2.T @ v
    wg = gt_W3.T @ w
    R = (vg[:, None, :] * wg[None, :, None]).reshape(256, 10)
    up = pi_W1.T @ u2
    vp = pi_W2.T @ v
    wp = pi_W3.T @ w
    VW = (vp[:, None, :] * wp[None, :, None]).reshape(625, 10)
    Cc = (jnp.einsum('abc,a,bo->o', ot_bias, att_W[:, 0], out_W)
          + 16.0 * att_b[0] * out_W.sum(axis=0)
          + 16.0 * out_b
          + jnp.einsum('ijk,i,jo,k->o', gt_bias, u1, v, w)
          + jnp.einsum('ijk,i,jo,k->o', pi_bias, u2, v, w)).reshape(1, 10)

    PI3 = batch_PI.reshape(_G, 5, 25, 2, 25, 2).transpose(
        0, 2, 4, 1, 3, 5).reshape(_G, 625, 20)
    Wc = conv_W.transpose(1, 2, 3, 0).reshape(20, 16)

    x3 = x.reshape(_G, _NPG, 128)

    degp = _sc_degree(dst)
    degp4 = degp[:, :_N].reshape(2, _G, _NPG, 16)
    p0, dinv = _tc_prep0(x3, gcn_W0, degp4)

    acc0 = _sc_scatter(p0.reshape(_N, 16), src, dst)
    h1, p1 = _tc_mid(
        acc0[:, :_N].reshape(2, _G, _NPG, 16), p0, dinv,
        gcn_b0.reshape(1, 16), mlp0_W1, mlp0_b1.reshape(1, 16), mlp0_W2,
        mlp0_b2.reshape(1, 256), gcn_W1)

    acc1 = _sc_scatter(p1.reshape(_N, 16), src, dst)
    score3 = _tc_final(
        acc1[:, :_N].reshape(2, _G, _NPG, 16), p1, dinv, h1,
        gcn_b1.reshape(1, 16), mlp1_W1, mlp1_b1.reshape(1, 16), mlp1_W2,
        mlp1_b2.reshape(1, 256), R, PI3, Wc, conv_b.reshape(1, 16),
        up.reshape(16, 1), VW, ug.reshape(1, 2), Cc)
    return score3.reshape(_G, 10)


# speedup vs baseline: 16.0528x; 1.4200x over previous
import functools

import jax
import jax.numpy as jnp
from jax import lax
from jax.experimental import pallas as pl
from jax.experimental.pallas import tpu as pltpu
from jax.experimental.pallas import tpu_sc as plsc

_N = 10000
_NPAD = 10240
_E = 160000
_CB = 128
_NCHUNK = _E // _CB
_NW = 32
_CPW = -(-_NCHUNK // _NW)
_RPW = _NPAD // 16
_ZCH = _RPW // _CB
_G = 8
_NPG = 1250
_F32 = jnp.float32


def _sc_mesh():
    return plsc.VectorSubcoreMesh(core_axis_name="c", subcore_axis_name="s")


def _sc_degree(dst2):
    @functools.partial(
        pl.kernel,
        out_type=jax.ShapeDtypeStruct((2, _NPAD, 16), _F32),
        mesh=_sc_mesh(),
        compiler_params=pltpu.CompilerParams(use_tc_tiling_on_sc=False),
        scratch_types=[
            pltpu.VMEM((_CPW, _CB), jnp.int32),
            pltpu.VMEM((_CB, 16), _F32),
            pltpu.VMEM_SHARED((_NPAD, 16), _F32),
            pltpu.SemaphoreType.DMA,
        ],
    )
    def k(dst_hbm, out_hbm, idx_v, val_v, acc_sh, ssem):
        c = lax.axis_index("c")
        s = lax.axis_index("s")
        wid = s * 2 + c
        nrows = jnp.where(wid < 2, 40, 39)
        base = 39 * wid + jnp.minimum(wid, 2)

        def fill_zero(i, carry):
            val_v[i, :] = jnp.zeros((16,), _F32)
            return carry

        lax.fori_loop(0, _CB, fill_zero, 0)
        for z in range(_ZCH):
            pltpu.sync_copy(val_v, acc_sh.at[pl.ds(s * _RPW + z * _CB, _CB)])

        pltpu.sync_copy(dst_hbm.at[pl.ds(base, 39)], idx_v.at[pl.ds(0, 39)])

        @pl.when(wid < 2)
        def _():
            pltpu.sync_copy(dst_hbm.at[pl.ds(base + 39, 1)],
                            idx_v.at[pl.ds(39, 1)])

        def fill_one(i, carry):
            val_v[i, :] = jnp.ones((16,), _F32)
            return carry

        lax.fori_loop(0, _CB, fill_one, 0)
        plsc.subcore_barrier()

        def fire(j, carry):
            pltpu.async_copy(val_v, acc_sh.at[idx_v.at[j]], ssem, add=True)
            return carry

        lax.fori_loop(0, nrows, fire, 0)

        def drain(j, carry):
            pltpu.make_async_copy(out_hbm.at[0, pl.ds(0, _CB)], val_v,
                                  ssem).wait()
            return carry

        lax.fori_loop(0, nrows, drain, 0)
        plsc.subcore_barrier()
        for z in range(_ZCH):
            pltpu.sync_copy(
                acc_sh.at[pl.ds(s * _RPW + z * _CB, _CB)],
                out_hbm.at[c, pl.ds(s * _RPW + z * _CB, _CB)],
            )

    return k(dst2)


_NB = 10
_NHALF = _CPW * 2 // _NB


def _sc_scatter(p, src2, dst2):
    @functools.partial(
        pl.kernel,
        out_type=jax.ShapeDtypeStruct((2, _NPAD, 16), _F32),
        mesh=_sc_mesh(),
        compiler_params=pltpu.CompilerParams(use_tc_tiling_on_sc=False),
        scratch_types=[
            pltpu.VMEM((_CPW, _CB), jnp.int32),
            pltpu.VMEM((_CPW, _CB), jnp.int32),
            pltpu.VMEM((2 * _NB, _CB, 16), _F32),
            pltpu.VMEM((_CB, 16), _F32),
            pltpu.VMEM_SHARED((_NPAD, 16), _F32),
            pltpu.SemaphoreType.DMA,
            pltpu.SemaphoreType.DMA,
            pltpu.SemaphoreType.DMA,
            pltpu.SemaphoreType.DMA,
        ],
    )
    def k(p_hbm, src_hbm, dst_hbm, out_hbm, sidx, didx, rows_v, zero_v,
          acc_sh, gsem0, gsem1, ssem0, ssem1):
        gsem = (gsem0, gsem1)
        ssem = (ssem0, ssem1)
        c = lax.axis_index("c")
        s = lax.axis_index("s")
        wid = s * 2 + c
        nrows = jnp.where(wid < 2, 40, 39)
        base = 39 * wid + jnp.minimum(wid, 2)

        def fill_zero(i, carry):
            zero_v[i, :] = jnp.zeros((16,), _F32)
            return carry

        lax.fori_loop(0, _CB, fill_zero, 0)
        for z in range(_ZCH):
            pltpu.sync_copy(zero_v, acc_sh.at[pl.ds(s * _RPW + z * _CB, _CB)])

        pltpu.sync_copy(src_hbm.at[pl.ds(base, 39)], sidx.at[pl.ds(0, 39)])
        pltpu.sync_copy(dst_hbm.at[pl.ds(base, 39)], didx.at[pl.ds(0, 39)])

        @pl.when(wid < 2)
        def _():
            pltpu.sync_copy(src_hbm.at[pl.ds(base + 39, 1)],
                            sidx.at[pl.ds(39, 1)])
            pltpu.sync_copy(dst_hbm.at[pl.ds(base + 39, 1)],
                            didx.at[pl.ds(39, 1)])
        plsc.subcore_barrier()

        def gather_fire(j, bi, S):
            pltpu.async_copy(p_hbm.at[sidx.at[j]], rows_v.at[bi], gsem[S])

        def drain_one(sem, bi):
            pltpu.make_async_copy(p_hbm.at[pl.ds(0, _CB)], rows_v.at[bi],
                                  sem).wait()

        def scat_fire(j, bi, S):
            pltpu.async_copy(rows_v.at[bi], acc_sh.at[didx.at[j]], ssem[S],
                             add=True)

        for b in range(_NB):
            gather_fire(jnp.int32(b), b, 0)

        def sg_body(sg, carry):
            for h in (0, 1):
                kh = sg * 2 + h
                S = h
                for b in range(_NB):
                    bio = (1 - S) * _NB + b
                    jprev = (kh - 1) * _NB + b
                    jnext = (kh + 1) * _NB + b

                    @pl.when(jnp.logical_and(kh >= 1, jprev < nrows))
                    def _():
                        drain_one(ssem[1 - S], bio)

                    @pl.when(jnext < nrows)
                    def _():
                        gather_fire(jnext, bio, 1 - S)
                for b in range(_NB):
                    bi = S * _NB + b
                    j = kh * _NB + b

                    @pl.when(j < nrows)
                    def _():
                        drain_one(gsem[S], bi)
                        scat_fire(j, bi, S)
            return carry

        lax.fori_loop(0, _NHALF // 2, sg_body, 0)

        for b in range(_NB):
            jlast = (_NHALF - 1) * _NB + b

            @pl.when(jlast < nrows)
            def _():
                drain_one(ssem[1], _NB + b)

        plsc.subcore_barrier()
        for z in range(_ZCH):
            pltpu.sync_copy(
                acc_sh.at[pl.ds(s * _RPW + z * _CB, _CB)],
                out_hbm.at[c, pl.ds(s * _RPW + z * _CB, _CB)],
            )

    return k(p, src2, dst2)


def _tc_prep0(x3, W0, degp4):
    def body(x_ref, w_ref, degp_ref, p0_ref, dinv_ref):
        deg = degp_ref[0, 0] + degp_ref[1, 0] + 1.0
        dinv = 1.0 / jnp.sqrt(deg[:, 0:1])
        h0 = jnp.dot(x_ref[0], w_ref[...], preferred_element_type=_F32,
                    precision=lax.Precision.HIGHEST)
        p0_ref[0] = h0 * dinv
        dinv_ref[0] = dinv

    return pl.pallas_call(
        body,
        grid=(_G,),
        in_specs=[
            pl.BlockSpec((1, _NPG, 128), lambda g: (g, 0, 0)),
            pl.BlockSpec((128, 16), lambda g: (0, 0)),
            pl.BlockSpec((2, 1, _NPG, 16), lambda g: (0, g, 0, 0)),
        ],
        out_specs=[
            pl.BlockSpec((1, _NPG, 16), lambda g: (g, 0, 0)),
            pl.BlockSpec((1, _NPG, 1), lambda g: (g, 0, 0)),
        ],
        out_shape=[
            jax.ShapeDtypeStruct((_G, _NPG, 16), _F32),
            jax.ShapeDtypeStruct((_G, _NPG, 1), _F32),
        ],
    )(x3, W0, degp4)


def _tc_mid(acc4, p0, dinv, b0, W1, b1, W2, b2, gW1):
    def body(acc_ref, p0_ref, dinv_ref, b0_ref, w1_ref, b1_ref, w2_ref,
             b2_ref, gw1_ref, h1_ref, p1_ref):
        dinv_b = dinv_ref[0]
        g0 = (acc_ref[0, 0] + acc_ref[1, 0] + p0_ref[0]) * dinv_b + b0_ref[...]
        t = jnp.maximum(
            jnp.dot(g0, w1_ref[...], preferred_element_type=_F32,
                    precision=lax.Precision.HIGHEST)
            + b1_ref[...], 0.0)
        h1 = jnp.dot(t, w2_ref[...], preferred_element_type=_F32,
                    precision=lax.Precision.HIGHEST) + b2_ref[...]
        h1_ref[0] = h1
        p1_ref[0] = jnp.dot(h1, gw1_ref[...],
                            preferred_element_type=_F32,
                    precision=lax.Precision.HIGHEST) * dinv_b

    full = lambda shape: pl.BlockSpec(shape, lambda g: tuple(0 for _ in shape))
    return pl.pallas_call(
        body,
        grid=(_G,),
        in_specs=[
            pl.BlockSpec((2, 1, _NPG, 16), lambda g: (0, g, 0, 0)),
            pl.BlockSpec((1, _NPG, 16), lambda g: (g, 0, 0)),
            pl.BlockSpec((1, _NPG, 1), lambda g: (g, 0, 0)),
            full((1, 16)),
            full((16, 16)),
            full((1, 16)),
            full((16, 256)),
            full((1, 256)),
            full((256, 16)),
        ],
        out_specs=[
            pl.BlockSpec((1, _NPG, 256), lambda g: (g, 0, 0)),
            pl.BlockSpec((1, _NPG, 16), lambda g: (g, 0, 0)),
        ],
        out_shape=[
            jax.ShapeDtypeStruct((_G, _NPG, 256), _F32),
            jax.ShapeDtypeStruct((_G, _NPG, 16), _F32),
        ],
    )(acc4, p0, dinv, b0, W1, b1, W2, b2, gW1)


def _tc_final(acc4, p1, dinv, h1, gb1, W1, b1, W2, b2, R, PI3, Wc, cb, up,
              VW, ug, Cc):
    def body(acc_ref, p1_ref, dinv_ref, h1_ref, gb1_ref, w1_ref, b1_ref,
             w2_ref, b2_ref, r_ref, pi_ref, wc_ref, cb_ref, up_ref, vw_ref,
             ug_ref, cc_ref, out_ref):
        dinv_b = dinv_ref[0]
        g1 = (acc_ref[0, 0] + acc_ref[1, 0] + p1_ref[0]) * dinv_b \
            + gb1_ref[...]
        t = jnp.maximum(
            jnp.dot(g1, w1_ref[...], preferred_element_type=_F32,
                    precision=lax.Precision.HIGHEST)
            + b1_ref[...], 0.0)
        h2 = jnp.dot(t, w2_ref[...], preferred_element_type=_F32,
                    precision=lax.Precision.HIGHEST) + b2_ref[...]
        s1 = jnp.sum(h1_ref[0], axis=0, keepdims=True) * (1.0 / _NPG)
        s2 = jnp.sum(h2, axis=0, keepdims=True) * (1.0 / _NPG)
        m1 = jnp.dot(s1, r_ref[...], preferred_element_type=_F32,
                    precision=lax.Precision.HIGHEST)
        m2 = jnp.dot(s2, r_ref[...], preferred_element_type=_F32,
                    precision=lax.Precision.HIGHEST)
        pe = jnp.maximum(
            jnp.dot(pi_ref[0], wc_ref[...], preferred_element_type=_F32,
                    precision=lax.Precision.HIGHEST)
            + cb_ref[...], 0.0)
        peu = jnp.dot(pe, up_ref[...], preferred_element_type=_F32,
                    precision=lax.Precision.HIGHEST)
        sp = jnp.sum(peu * vw_ref[...], axis=0, keepdims=True)
        out_ref[0] = ug_ref[0, 0] * m1 + ug_ref[0, 1] * m2 + sp + cc_ref[...]

    full = lambda shape: pl.BlockSpec(shape, lambda g: tuple(0 for _ in shape))
    return pl.pallas_call(
        body,
        grid=(_G,),
        in_specs=[
            pl.BlockSpec((2, 1, _NPG, 16), lambda g: (0, g, 0, 0)),
            pl.BlockSpec((1, _NPG, 16), lambda g: (g, 0, 0)),
            pl.BlockSpec((1, _NPG, 1), lambda g: (g, 0, 0)),
            pl.BlockSpec((1, _NPG, 256), lambda g: (g, 0, 0)),
            full((1, 16)),
            full((16, 16)),
            full((1, 16)),
            full((16, 256)),
            full((1, 256)),
            full((256, 10)),
            pl.BlockSpec((1, 625, 20), lambda g: (g, 0, 0)),
            full((20, 16)),
            full((1, 16)),
            full((16, 1)),
            full((625, 10)),
            full((1, 2)),
            full((1, 10)),
        ],
        out_specs=pl.BlockSpec((1, 1, 10), lambda g: (g, 0, 0)),
        out_shape=jax.ShapeDtypeStruct((_G, 1, 10), _F32),
    )(acc4, p1, dinv, h1, gb1, W1, b1, W2, b2, R, PI3, Wc, cb, up, VW, ug, Cc)


def kernel(x, edge_index, batch_PI, gcn_W0, gcn_b0, mlp0_W1, mlp0_b1,
           mlp0_W2, mlp0_b2, gcn_W1, gcn_b1, mlp1_W1, mlp1_b1, mlp1_W2,
           mlp1_b2, conv_W, conv_b, pi_W1, pi_W2, pi_W3, pi_bias, gt_W1,
           gt_W2, gt_W3, gt_bias, ot_W1, ot_W2, ot_W3, ot_bias, att_W,
           att_b, out_W, out_b):
    src = edge_index[0]
    dst = edge_index[1]

    u = ot_W1.T @ att_W[:, 0]
    v = ot_W2.T @ out_W
    w = ot_W3.sum(axis=0)
    u1, u2 = u[:16], u[16:]
    ug = gt_W1.T @ u1
    vg = gt_W2.T @ v
    wg = gt_W3.T @ w
    R = (vg[:, None, :] * wg[None, :, None]).reshape(256, 10)
    up = pi_W1.T @ u2
    vp = pi_W2.T @ v
    wp = pi_W3.T @ w
    VW = (vp[:, None, :] * wp[None, :, None]).reshape(625, 10)
    Cc = (jnp.einsum('abc,a,bo->o', ot_bias, att_W[:, 0], out_W)
          + 16.0 * att_b[0] * out_W.sum(axis=0)
          + 16.0 * out_b
          + jnp.einsum('ijk,i,jo,k->o', gt_bias, u1, v, w)
          + jnp.einsum('ijk,i,jo,k->o', pi_bias, u2, v, w)).reshape(1, 10)

    PI3 = batch_PI.reshape(_G, 5, 25, 2, 25, 2).transpose(
        0, 2, 4, 1, 3, 5).reshape(_G, 625, 20)
    Wc = conv_W.transpose(1, 2, 3, 0).reshape(20, 16)

    x3 = x.reshape(_G, _NPG, 128)

    src2 = src.reshape(_NCHUNK, _CB)
    dst2 = dst.reshape(_NCHUNK, _CB)
    degp = _sc_degree(dst2)
    degp4 = degp[:, :_N].reshape(2, _G, _NPG, 16)
    p0, dinv = _tc_prep0(x3, gcn_W0, degp4)

    acc0 = _sc_scatter(p0.reshape(_N, 16), src2, dst2)
    h1, p1 = _tc_mid(
        acc0[:, :_N].reshape(2, _G, _NPG, 16), p0, dinv,
        gcn_b0.reshape(1, 16), mlp0_W1, mlp0_b1.reshape(1, 16), mlp0_W2,
        mlp0_b2.reshape(1, 256), gcn_W1)

    acc1 = _sc_scatter(p1.reshape(_N, 16), src2, dst2)
    score3 = _tc_final(
        acc1[:, :_N].reshape(2, _G, _NPG, 16), p1, dinv, h1,
        gcn_b1.reshape(1, 16), mlp1_W1, mlp1_b1.reshape(1, 16), mlp1_W2,
        mlp1_b2.reshape(1, 256), R, PI3, Wc, conv_b.reshape(1, 16),
        up.reshape(16, 1), VW, ug.reshape(1, 2), Cc)
    return score3.reshape(_G, 10)


# speedup vs baseline: 18.2995x; 1.1400x over previous
import functools

import jax
import jax.numpy as jnp
from jax import lax
from jax.experimental import pallas as pl
from jax.experimental.pallas import tpu as pltpu
from jax.experimental.pallas import tpu_sc as plsc

_N = 10000
_NPAD = 10000
_E = 160000
_CB = 128
_NCHUNK = _E // _CB
_NW = 32
_CPW = -(-_NCHUNK // _NW)
_RPW = _NPAD // 16
_ZCHUNKS = ((0, 128), (128, 128), (256, 128), (384, 128), (512, 97), (609, 16))
_G = 8
_NPG = 1250
_F32 = jnp.float32


def _sc_mesh():
    return plsc.VectorSubcoreMesh(core_axis_name="c", subcore_axis_name="s")


def _sc_degree(dst2):
    @functools.partial(
        pl.kernel,
        out_type=jax.ShapeDtypeStruct((2, _NPAD, 16), _F32),
        mesh=_sc_mesh(),
        compiler_params=pltpu.CompilerParams(use_tc_tiling_on_sc=False),
        scratch_types=[
            pltpu.VMEM((_CPW, _CB), jnp.int32),
            pltpu.VMEM((_CB, 16), _F32),
            pltpu.VMEM_SHARED((_NPAD, 16), _F32),
            pltpu.SemaphoreType.DMA,
        ],
    )
    def k(dst_hbm, out_hbm, idx_v, val_v, acc_sh, ssem):
        c = lax.axis_index("c")
        s = lax.axis_index("s")
        wid = s * 2 + c
        nrows = jnp.where(wid < 2, 40, 39)
        base = 39 * wid + jnp.minimum(wid, 2)

        def fill_zero(i, carry):
            val_v[i, :] = jnp.zeros((16,), _F32)
            return carry

        lax.fori_loop(0, _CB, fill_zero, 0)
        for off, sz in _ZCHUNKS:
            pltpu.sync_copy(val_v.at[pl.ds(0, sz)],
                            acc_sh.at[pl.ds(s * _RPW + off, sz)])

        pltpu.sync_copy(dst_hbm.at[pl.ds(base, 39)], idx_v.at[pl.ds(0, 39)])

        @pl.when(wid < 2)
        def _():
            pltpu.sync_copy(dst_hbm.at[pl.ds(base + 39, 1)],
                            idx_v.at[pl.ds(39, 1)])

        def fill_one(i, carry):
            val_v[i, :] = jnp.ones((16,), _F32)
            return carry

        lax.fori_loop(0, _CB, fill_one, 0)
        plsc.subcore_barrier()

        def fire(j, carry):
            pltpu.async_copy(val_v, acc_sh.at[idx_v.at[j]], ssem, add=True)
            return carry

        lax.fori_loop(0, nrows, fire, 0)

        def drain(j, carry):
            pltpu.make_async_copy(out_hbm.at[0, pl.ds(0, _CB)], val_v,
                                  ssem).wait()
            return carry

        lax.fori_loop(0, nrows, drain, 0)
        plsc.subcore_barrier()
        for off, sz in _ZCHUNKS:
            pltpu.sync_copy(
                acc_sh.at[pl.ds(s * _RPW + off, sz)],
                out_hbm.at[c, pl.ds(s * _RPW + off, sz)],
            )

    return k(dst2)


_NB = 10
_NHALF = _CPW * 2 // _NB


def _sc_scatter(p, src2, dst2):
    @functools.partial(
        pl.kernel,
        out_type=jax.ShapeDtypeStruct((2, _NPAD, 16), _F32),
        mesh=_sc_mesh(),
        compiler_params=pltpu.CompilerParams(use_tc_tiling_on_sc=False),
        scratch_types=[
            pltpu.VMEM((_CPW, _CB), jnp.int32),
            pltpu.VMEM((_CPW, _CB), jnp.int32),
            pltpu.VMEM((2 * _NB, _CB, 16), _F32),
            pltpu.VMEM((_CB, 16), _F32),
            pltpu.VMEM_SHARED((_NPAD, 16), _F32),
            pltpu.SemaphoreType.DMA,
            pltpu.SemaphoreType.DMA,
            pltpu.SemaphoreType.DMA,
            pltpu.SemaphoreType.DMA,
        ],
    )
    def k(p_hbm, src_hbm, dst_hbm, out_hbm, sidx, didx, rows_v, zero_v,
          acc_sh, gsem0, gsem1, ssem0, ssem1):
        gsem = (gsem0, gsem1)
        ssem = (ssem0, ssem1)
        c = lax.axis_index("c")
        s = lax.axis_index("s")
        wid = s * 2 + c
        nrows = jnp.where(wid < 2, 40, 39)
        base = 39 * wid + jnp.minimum(wid, 2)

        def fill_zero(i, carry):
            zero_v[i, :] = jnp.zeros((16,), _F32)
            return carry

        lax.fori_loop(0, _CB, fill_zero, 0)
        for off, sz in _ZCHUNKS:
            pltpu.sync_copy(zero_v.at[pl.ds(0, sz)],
                            acc_sh.at[pl.ds(s * _RPW + off, sz)])

        pltpu.sync_copy(src_hbm.at[pl.ds(base, 39)], sidx.at[pl.ds(0, 39)])
        pltpu.sync_copy(dst_hbm.at[pl.ds(base, 39)], didx.at[pl.ds(0, 39)])

        @pl.when(wid < 2)
        def _():
            pltpu.sync_copy(src_hbm.at[pl.ds(base + 39, 1)],
                            sidx.at[pl.ds(39, 1)])
            pltpu.sync_copy(dst_hbm.at[pl.ds(base + 39, 1)],
                            didx.at[pl.ds(39, 1)])
        plsc.subcore_barrier()

        def gather_fire(j, bi, S):
            pltpu.async_copy(p_hbm.at[sidx.at[j]], rows_v.at[bi], gsem[S])

        def drain_one(sem, bi):
            pltpu.make_async_copy(p_hbm.at[pl.ds(0, _CB)], rows_v.at[bi],
                                  sem).wait()

        def scat_fire(j, bi, S):
            pltpu.async_copy(rows_v.at[bi], acc_sh.at[didx.at[j]], ssem[S],
                             add=True)

        for b in range(_NB):
            gather_fire(jnp.int32(b), b, 0)

        def sg_body(sg, carry):
            for h in (0, 1):
                kh = sg * 2 + h
                S = h
                for b in range(_NB):
                    bio = (1 - S) * _NB + b
                    jprev = (kh - 1) * _NB + b
                    jnext = (kh + 1) * _NB + b

                    @pl.when(jnp.logical_and(kh >= 1, jprev < nrows))
                    def _():
                        drain_one(ssem[1 - S], bio)

                    @pl.when(jnext < nrows)
                    def _():
                        gather_fire(jnext, bio, 1 - S)
                for b in range(_NB):
                    bi = S * _NB + b
                    j = kh * _NB + b

                    @pl.when(j < nrows)
                    def _():
                        drain_one(gsem[S], bi)
                        scat_fire(j, bi, S)
            return carry

        lax.fori_loop(0, _NHALF // 2, sg_body, 0)

        for b in range(_NB):
            jlast = (_NHALF - 1) * _NB + b

            @pl.when(jlast < nrows)
            def _():
                drain_one(ssem[1], _NB + b)

        plsc.subcore_barrier()
        for off, sz in _ZCHUNKS:
            pltpu.sync_copy(
                acc_sh.at[pl.ds(s * _RPW + off, sz)],
                out_hbm.at[c, pl.ds(s * _RPW + off, sz)],
            )

    return k(p, src2, dst2)


def _tc_prep0(x3, W0, degp4):
    def body(x_ref, w_ref, degp_ref, p0_ref, dinv_ref):
        deg = degp_ref[0, 0] + degp_ref[1, 0] + 1.0
        dinv = 1.0 / jnp.sqrt(deg[:, 0:1])
        h0 = jnp.dot(x_ref[0], w_ref[...], preferred_element_type=_F32,
                    precision=lax.Precision.HIGHEST)
        p0_ref[0] = h0 * dinv
        dinv_ref[0] = dinv

    return pl.pallas_call(
        body,
        grid=(_G,),
        in_specs=[
            pl.BlockSpec((1, _NPG, 128), lambda g: (g, 0, 0)),
            pl.BlockSpec((128, 16), lambda g: (0, 0)),
            pl.BlockSpec((2, 1, _NPG, 16), lambda g: (0, g, 0, 0)),
        ],
        out_specs=[
            pl.BlockSpec((1, _NPG, 16), lambda g: (g, 0, 0)),
            pl.BlockSpec((1, _NPG, 1), lambda g: (g, 0, 0)),
        ],
        out_shape=[
            jax.ShapeDtypeStruct((_G, _NPG, 16), _F32),
            jax.ShapeDtypeStruct((_G, _NPG, 1), _F32),
        ],
    )(x3, W0, degp4)


def _tc_mid(acc4, p0, dinv, b0, W1, b1, W2, b2, gW1):
    def body(acc_ref, p0_ref, dinv_ref, b0_ref, w1_ref, b1_ref, w2_ref,
             b2_ref, gw1_ref, h1_ref, p1_ref):
        dinv_b = dinv_ref[0]
        g0 = (acc_ref[0, 0] + acc_ref[1, 0] + p0_ref[0]) * dinv_b + b0_ref[...]
        t = jnp.maximum(
            jnp.dot(g0, w1_ref[...], preferred_element_type=_F32,
                    precision=lax.Precision.HIGHEST)
            + b1_ref[...], 0.0)
        h1 = jnp.dot(t, w2_ref[...], preferred_element_type=_F32,
                    precision=lax.Precision.HIGHEST) + b2_ref[...]
        h1_ref[0] = h1
        p1_ref[0] = jnp.dot(h1, gw1_ref[...],
                            preferred_element_type=_F32,
                    precision=lax.Precision.HIGHEST) * dinv_b

    full = lambda shape: pl.BlockSpec(shape, lambda g: tuple(0 for _ in shape))
    return pl.pallas_call(
        body,
        grid=(_G,),
        in_specs=[
            pl.BlockSpec((2, 1, _NPG, 16), lambda g: (0, g, 0, 0)),
            pl.BlockSpec((1, _NPG, 16), lambda g: (g, 0, 0)),
            pl.BlockSpec((1, _NPG, 1), lambda g: (g, 0, 0)),
            full((1, 16)),
            full((16, 16)),
            full((1, 16)),
            full((16, 256)),
            full((1, 256)),
            full((256, 16)),
        ],
        out_specs=[
            pl.BlockSpec((1, _NPG, 256), lambda g: (g, 0, 0)),
            pl.BlockSpec((1, _NPG, 16), lambda g: (g, 0, 0)),
        ],
        out_shape=[
            jax.ShapeDtypeStruct((_G, _NPG, 256), _F32),
            jax.ShapeDtypeStruct((_G, _NPG, 16), _F32),
        ],
    )(acc4, p0, dinv, b0, W1, b1, W2, b2, gW1)


def _tc_final(acc4, p1, dinv, h1, gb1, W1, b1, W2, b2, R, PI3, Wc, cb, up,
              VW, ug, Cc):
    def body(acc_ref, p1_ref, dinv_ref, h1_ref, gb1_ref, w1_ref, b1_ref,
             w2_ref, b2_ref, r_ref, pi_ref, wc_ref, cb_ref, up_ref, vw_ref,
             ug_ref, cc_ref, out_ref):
        dinv_b = dinv_ref[0]
        g1 = (acc_ref[0, 0] + acc_ref[1, 0] + p1_ref[0]) * dinv_b \
            + gb1_ref[...]
        t = jnp.maximum(
            jnp.dot(g1, w1_ref[...], preferred_element_type=_F32,
                    precision=lax.Precision.HIGHEST)
            + b1_ref[...], 0.0)
        h2 = jnp.dot(t, w2_ref[...], preferred_element_type=_F32,
                    precision=lax.Precision.HIGHEST) + b2_ref[...]
        s1 = jnp.sum(h1_ref[0], axis=0, keepdims=True) * (1.0 / _NPG)
        s2 = jnp.sum(h2, axis=0, keepdims=True) * (1.0 / _NPG)
        m1 = jnp.dot(s1, r_ref[...], preferred_element_type=_F32,
                    precision=lax.Precision.HIGHEST)
        m2 = jnp.dot(s2, r_ref[...], preferred_element_type=_F32,
                    precision=lax.Precision.HIGHEST)
        pe = jnp.maximum(
            jnp.dot(pi_ref[0], wc_ref[...], preferred_element_type=_F32,
                    precision=lax.Precision.HIGHEST)
            + cb_ref[...], 0.0)
        peu = jnp.dot(pe, up_ref[...], preferred_element_type=_F32,
                    precision=lax.Precision.HIGHEST)
        sp = jnp.sum(peu * vw_ref[...], axis=0, keepdims=True)
        out_ref[0] = ug_ref[0, 0] * m1 + ug_ref[0, 1] * m2 + sp + cc_ref[...]

    full = lambda shape: pl.BlockSpec(shape, lambda g: tuple(0 for _ in shape))
    return pl.pallas_call(
        body,
        grid=(_G,),
        in_specs=[
            pl.BlockSpec((2, 1, _NPG, 16), lambda g: (0, g, 0, 0)),
            pl.BlockSpec((1, _NPG, 16), lambda g: (g, 0, 0)),
            pl.BlockSpec((1, _NPG, 1), lambda g: (g, 0, 0)),
            pl.BlockSpec((1, _NPG, 256), lambda g: (g, 0, 0)),
            full((1, 16)),
            full((16, 16)),
            full((1, 16)),
            full((16, 256)),
            full((1, 256)),
            full((256, 10)),
            pl.BlockSpec((1, 625, 20), lambda g: (g, 0, 0)),
            full((20, 16)),
            full((1, 16)),
            full((16, 1)),
            full((625, 10)),
            full((1, 2)),
            full((1, 10)),
        ],
        out_specs=pl.BlockSpec((1, 1, 10), lambda g: (g, 0, 0)),
        out_shape=jax.ShapeDtypeStruct((_G, 1, 10), _F32),
    )(acc4, p1, dinv, h1, gb1, W1, b1, W2, b2, R, PI3, Wc, cb, up, VW, ug, Cc)


def kernel(x, edge_index, batch_PI, gcn_W0, gcn_b0, mlp0_W1, mlp0_b1,
           mlp0_W2, mlp0_b2, gcn_W1, gcn_b1, mlp1_W1, mlp1_b1, mlp1_W2,
           mlp1_b2, conv_W, conv_b, pi_W1, pi_W2, pi_W3, pi_bias, gt_W1,
           gt_W2, gt_W3, gt_bias, ot_W1, ot_W2, ot_W3, ot_bias, att_W,
           att_b, out_W, out_b):
    src = edge_index[0]
    dst = edge_index[1]

    u = ot_W1.T @ att_W[:, 0]
    v = ot_W2.T @ out_W
    w = ot_W3.sum(axis=0)
    u1, u2 = u[:16], u[16:]
    ug = gt_W1.T @ u1
    vg = gt_W2.T @ v
    wg = gt_W3.T @ w
    R = (vg[:, None, :] * wg[None, :, None]).reshape(256, 10)
    up = pi_W1.T @ u2
    vp = pi_W2.T @ v
    wp = pi_W3.T @ w
    VW = (vp[:, None, :] * wp[None, :, None]).reshape(625, 10)
    Cc = (jnp.einsum('abc,a,bo->o', ot_bias, att_W[:, 0], out_W)
          + 16.0 * att_b[0] * out_W.sum(axis=0)
          + 16.0 * out_b
          + jnp.einsum('ijk,i,jo,k->o', gt_bias, u1, v, w)
          + jnp.einsum('ijk,i,jo,k->o', pi_bias, u2, v, w)).reshape(1, 10)

    PI3 = batch_PI.reshape(_G, 5, 25, 2, 25, 2).transpose(
        0, 2, 4, 1, 3, 5).reshape(_G, 625, 20)
    Wc = conv_W.transpose(1, 2, 3, 0).reshape(20, 16)

    x3 = x.reshape(_G, _NPG, 128)

    src2 = src.reshape(_NCHUNK, _CB)
    dst2 = dst.reshape(_NCHUNK, _CB)
    degp = _sc_degree(dst2)
    degp4 = degp.reshape(2, _G, _NPG, 16)
    p0, dinv = _tc_prep0(x3, gcn_W0, degp4)

    acc0 = _sc_scatter(p0.reshape(_N, 16), src2, dst2)
    h1, p1 = _tc_mid(
        acc0.reshape(2, _G, _NPG, 16), p0, dinv,
        gcn_b0.reshape(1, 16), mlp0_W1, mlp0_b1.reshape(1, 16), mlp0_W2,
        mlp0_b2.reshape(1, 256), gcn_W1)

    acc1 = _sc_scatter(p1.reshape(_N, 16), src2, dst2)
    score3 = _tc_final(
        acc1.reshape(2, _G, _NPG, 16), p1, dinv, h1,
        gcn_b1.reshape(1, 16), mlp1_W1, mlp1_b1.reshape(1, 16), mlp1_W2,
        mlp1_b2.reshape(1, 256), R, PI3, Wc, conv_b.reshape(1, 16),
        up.reshape(16, 1), VW, ug.reshape(1, 2), Cc)
    return score3.reshape(_G, 10)
